# baseline (speedup 1.0000x reference)
import jax
import jax.numpy as jnp
from jax import lax
from jax.experimental import pallas as pl
from jax.experimental.pallas import tpu as pltpu
from jax.experimental.pallas import tpu_sc as plsc

N = 50000
E = 800000
EG = 300000
DIN = 3
H = 64
HH = 32
LW = 128
NC, NS = 2, 16
NW = NC * NS
EDGE_HID = 4 * (2 * H + 2)

NP = 51200
ROWS_PER_TILE = NP // NS
DUMMY = N

CH = 128
CG = 64
EG_CHUNKS = 294
EGP = NS * CG * EG_CHUNKS
EGT = CG * EG_CHUNKS

NSLAB = 4
E_CHUNKS = 49
EPS = NW * CH * E_CHUNKS
EP = NSLAB * EPS
EPW = CH * E_CHUNKS

_MESH = plsc.VectorSubcoreMesh(core_axis_name="c", subcore_axis_name="s")
_SC_PARAMS = pltpu.CompilerParams(needs_layout_passes=False,
                                  use_tc_tiling_on_sc=False)


def _make_gat_edge_kernel():
    def body(htab, srcg, dstg, ad, num_out, den_out,
             src0, dst0, src1, dst1, ex_b, ad0, ad1, rows0, rows1, rows32_b,
             acc_sh, den_sh, ad_sh, semr0, semr1, sema0, sema1):
        c = lax.axis_index("c")
        s = lax.axis_index("s")
        r0 = s * ROWS_PER_TILE

        pltpu.sync_copy(ad.at[pl.ds(r0, ROWS_PER_TILE)],
                        ad_sh.at[pl.ds(r0, ROWS_PER_TILE)])

        zeros16 = jnp.zeros((16,), jnp.float32)
        for j in range(CG):
            rows32_b[j, pl.ds(0, 16)] = zeros16
            rows32_b[j, pl.ds(16, 16)] = zeros16
        for i in range(CG // 16):
            ex_b[pl.ds(i * 16, 16)] = zeros16
        nfull = ROWS_PER_TILE // CG
        for k in range(nfull):
            pltpu.sync_copy(rows32_b, acc_sh.at[pl.ds(r0 + k * CG, CG)])

        @pl.when(c == 0)
        def _zero_den():
            for k in range(nfull):
                pltpu.sync_copy(ex_b, den_sh.at[pl.ds(r0 + k * CG, CG)])

        plsc.subcore_barrier()

        ebase = s * EGT
        iota16 = lax.iota(jnp.int32, 16)
        col64 = jnp.full((16,), 2 * HH, jnp.int32)

        def fire(g, sb, db, rb, ab, semr, sema):
            e0 = ebase + g * CG
            pltpu.sync_copy(srcg.at[pl.ds(e0, CG)], sb)
            pltpu.sync_copy(dstg.at[pl.ds(e0, CG)], db)
            pltpu.async_copy(htab.at[sb], rb, semr)
            pltpu.async_copy(ad_sh.at[db], ab, sema)

        def process(sb, db, rb, ab, semr, sema):
            pltpu.make_async_copy(htab.at[sb], rb, semr).wait()
            pltpu.make_async_copy(ad_sh.at[db], ab, sema).wait()
            for i in range(CG // 16):
                av = plsc.load_gather(rb, [iota16 + (i * 16), col64])
                bv = ab[pl.ds(i * 16, 16)]
                e = av + bv
                e = jnp.where(e >= 0.0, e, 0.2 * e)
                ex_b[pl.ds(i * 16, 16)] = jnp.exp(e)

            def scale(off):
                for i in range(CG // 16):
                    ev = ex_b[pl.ds(i * 16, 16)]
                    for k in range(16):
                        j = i * 16 + k
                        m = ev[k]
                        rows32_b[j, pl.ds(0, 16)] = rb[j, pl.ds(off, 16)] * m
                        rows32_b[j, pl.ds(16, 16)] = \
                            rb[j, pl.ds(off + 16, 16)] * m

            @pl.when(c == 0)
            def _scale0():
                scale(0)

            @pl.when(c == 1)
            def _scale1():
                scale(HH)

            pltpu.sync_copy(rows32_b, acc_sh.at[db], add=True)

            @pl.when(c == 0)
            def _add_den():
                pltpu.sync_copy(ex_b, den_sh.at[db], add=True)

        fire(0, src0, dst0, rows0, ad0, semr0, sema0)

        def pair(t, carry):
            g0 = 2 * t
            fire(g0 + 1, src1, dst1, rows1, ad1, semr1, sema1)
            process(src0, dst0, rows0, ad0, semr0, sema0)
            g2 = jnp.where(g0 + 2 < EG_CHUNKS, g0 + 2, 0)
            fire(g2, src0, dst0, rows0, ad0, semr0, sema0)
            process(src1, dst1, rows1, ad1, semr1, sema1)
            return carry

        lax.fori_loop(0, EG_CHUNKS // 2, pair, 0)
        pltpu.make_async_copy(htab.at[src0], rows0, semr0).wait()
        pltpu.make_async_copy(ad_sh.at[dst0], ad0, sema0).wait()
        plsc.subcore_barrier()

        pltpu.sync_copy(acc_sh.at[pl.ds(r0, ROWS_PER_TILE)],
                        num_out.at[c, pl.ds(r0, ROWS_PER_TILE)])

        @pl.when(c == 0)
        def _write_den():
            pltpu.sync_copy(den_sh.at[pl.ds(r0, ROWS_PER_TILE)],
                            den_out.at[pl.ds(r0, ROWS_PER_TILE)])

    return pl.kernel(
        body,
        out_type=[
            jax.ShapeDtypeStruct((NC, NP, HH), jnp.float32),
            jax.ShapeDtypeStruct((NP,), jnp.float32),
        ],
        mesh=_MESH,
        scratch_types=[
            pltpu.VMEM((CG,), jnp.int32),
            pltpu.VMEM((CG,), jnp.int32),
            pltpu.VMEM((CG,), jnp.int32),
            pltpu.VMEM((CG,), jnp.int32),
            pltpu.VMEM((CG,), jnp.float32),
            pltpu.VMEM((CG,), jnp.float32),
            pltpu.VMEM((CG,), jnp.float32),
            pltpu.VMEM((CG, LW), jnp.float32),
            pltpu.VMEM((CG, LW), jnp.float32),
            pltpu.VMEM((CG, HH), jnp.float32),
            pltpu.VMEM_SHARED((NP, HH), jnp.float32),
            pltpu.VMEM_SHARED((NP,), jnp.float32),
            pltpu.VMEM_SHARED((NP,), jnp.float32),
            pltpu.SemaphoreType.DMA,
            pltpu.SemaphoreType.DMA,
            pltpu.SemaphoreType.DMA,
            pltpu.SemaphoreType.DMA,
        ],
        compiler_params=_SC_PARAMS,
    )


_gat_edge = _make_gat_edge_kernel()


def _make_edge_gather_kernel():
    def body(h2tab, ei0, ei1, es_out, ed_out,
             is0, id0, is1, id1, rs0, rd0, rs1, rd1,
             ss0, sd0, ss1, sd1):
        c = lax.axis_index("c")
        s = lax.axis_index("s")
        wid = s * NC + c
        base = wid * EPW

        def fire(g, ib, jb, rb, qb, sems, semd):
            e0 = base + g * CH
            pltpu.sync_copy(ei0.at[pl.ds(e0, CH)], ib)
            pltpu.sync_copy(ei1.at[pl.ds(e0, CH)], jb)
            pltpu.async_copy(h2tab.at[ib], rb, sems)
            pltpu.async_copy(h2tab.at[jb], qb, semd)

        def process(g, ib, jb, rb, qb, sems, semd):
            e0 = base + g * CH
            pltpu.make_async_copy(h2tab.at[ib], rb, sems).wait()
            pltpu.sync_copy(rb.at[pl.ds(0, CH), pl.ds(0, H)],
                            es_out.at[pl.ds(e0, CH)])
            pltpu.make_async_copy(h2tab.at[jb], qb, semd).wait()
            pltpu.sync_copy(qb.at[pl.ds(0, CH), pl.ds(0, H)],
                            ed_out.at[pl.ds(e0, CH)])

        fire(0, is0, id0, rs0, rd0, ss0, sd0)

        def pair(t, carry):
            g0 = 2 * t
            fire(g0 + 1, is1, id1, rs1, rd1, ss1, sd1)
            process(g0, is0, id0, rs0, rd0, ss0, sd0)
            fire(g0 + 2, is0, id0, rs0, rd0, ss0, sd0)
            process(g0 + 1, is1, id1, rs1, rd1, ss1, sd1)
            return carry

        lax.fori_loop(0, E_CHUNKS // 2, pair, 0)
        process(E_CHUNKS - 1, is0, id0, rs0, rd0, ss0, sd0)

    return pl.kernel(
        body,
        out_type=[
            jax.ShapeDtypeStruct((EPS, H), jnp.float32),
            jax.ShapeDtypeStruct((EPS, H), jnp.float32),
        ],
        mesh=_MESH,
        scratch_types=[
            pltpu.VMEM((CH,), jnp.int32),
            pltpu.VMEM((CH,), jnp.int32),
            pltpu.VMEM((CH,), jnp.int32),
            pltpu.VMEM((CH,), jnp.int32),
            pltpu.VMEM((CH, LW), jnp.float32),
            pltpu.VMEM((CH, LW), jnp.float32),
            pltpu.VMEM((CH, LW), jnp.float32),
            pltpu.VMEM((CH, LW), jnp.float32),
            pltpu.SemaphoreType.DMA,
            pltpu.SemaphoreType.DMA,
            pltpu.SemaphoreType.DMA,
            pltpu.SemaphoreType.DMA,
        ],
        compiler_params=_SC_PARAMS,
    )


_edge_gather = _make_edge_gather_kernel()


def _tc_a_body(x_ref, w_ref, asr_ref, adr_ref, ht_ref, ad_ref):
    h = jnp.dot(x_ref[...], w_ref[...], preferred_element_type=jnp.float32)
    a_s = jnp.dot(h, asr_ref[...], preferred_element_type=jnp.float32)
    pad = jnp.zeros((h.shape[0], LW - H - 1), jnp.float32)
    ht_ref[...] = jnp.concatenate([h, a_s, pad], axis=1)
    ad_ref[...] = jnp.dot(h, adr_ref[...], preferred_element_type=jnp.float32)


def _tc_a(x_pad, W1, a_src, a_dst):
    BN = 6400
    nb = NP // BN
    return pl.pallas_call(
        _tc_a_body,
        grid=(nb,),
        in_specs=[
            pl.BlockSpec((BN, DIN), lambda i: (i, 0)),
            pl.BlockSpec((DIN, H), lambda i: (0, 0)),
            pl.BlockSpec((H, 1), lambda i: (0, 0)),
            pl.BlockSpec((H, 1), lambda i: (0, 0)),
        ],
        out_specs=[
            pl.BlockSpec((BN, LW), lambda i: (i, 0)),
            pl.BlockSpec((BN, 1), lambda i: (i, 0)),
        ],
        out_shape=[
            jax.ShapeDtypeStruct((NP, LW), jnp.float32),
            jax.ShapeDtypeStruct((NP, 1), jnp.float32),
        ],
    )(x_pad, W1, a_src.reshape(H, 1), a_dst.reshape(H, 1))


def _tc_c1_body(num_ref, den_ref, b_ref, w_ref, asr_ref, adr_ref,
                ht_ref, ad_ref):
    acc = jnp.concatenate([num_ref[0], num_ref[1]], axis=1)
    h = acc / (den_ref[...] + 1e-16) + b_ref[...]
    h = jnp.where(h > 0.0, h, jnp.exp(h) - 1.0)
    hlin = jnp.dot(h, w_ref[...], preferred_element_type=jnp.float32)
    a_s = jnp.dot(hlin, asr_ref[...], preferred_element_type=jnp.float32)
    pad = jnp.zeros((hlin.shape[0], LW - H - 1), jnp.float32)
    ht_ref[...] = jnp.concatenate([hlin, a_s, pad], axis=1)
    ad_ref[...] = jnp.dot(hlin, adr_ref[...], preferred_element_type=jnp.float32)


def _tc_c1(num, den, b, W2, a_src, a_dst):
    BN = 6400
    nb = NP // BN
    return pl.pallas_call(
        _tc_c1_body,
        grid=(nb,),
        in_specs=[
            pl.BlockSpec((2, BN, HH), lambda i: (0, i, 0)),
            pl.BlockSpec((BN, 1), lambda i: (i, 0)),
            pl.BlockSpec((1, H), lambda i: (0, 0)),
            pl.BlockSpec((H, H), lambda i: (0, 0)),
            pl.BlockSpec((H, 1), lambda i: (0, 0)),
            pl.BlockSpec((H, 1), lambda i: (0, 0)),
        ],
        out_specs=[
            pl.BlockSpec((BN, LW), lambda i: (i, 0)),
            pl.BlockSpec((BN, 1), lambda i: (i, 0)),
        ],
        out_shape=[
            jax.ShapeDtypeStruct((NP, LW), jnp.float32),
            jax.ShapeDtypeStruct((NP, 1), jnp.float32),
        ],
    )(num, den.reshape(NP, 1), b.reshape(1, H), W2,
      a_src.reshape(H, 1), a_dst.reshape(H, 1))


def _tc_c2_body(num_ref, den_ref, b_ref, ht_ref):
    acc = jnp.concatenate([num_ref[0], num_ref[1]], axis=1)
    h = acc / (den_ref[...] + 1e-16) + b_ref[...]
    h = jnp.where(h > 0.0, h, jnp.exp(h) - 1.0)
    pad = jnp.zeros((h.shape[0], LW - H), jnp.float32)
    ht_ref[...] = jnp.concatenate([h, pad], axis=1)


def _tc_c2(num, den, b):
    BN = 6400
    nb = NP // BN
    return pl.pallas_call(
        _tc_c2_body,
        grid=(nb,),
        in_specs=[
            pl.BlockSpec((2, BN, HH), lambda i: (0, i, 0)),
            pl.BlockSpec((BN, 1), lambda i: (i, 0)),
            pl.BlockSpec((1, H), lambda i: (0, 0)),
        ],
        out_specs=pl.BlockSpec((BN, LW), lambda i: (i, 0)),
        out_shape=jax.ShapeDtypeStruct((NP, LW), jnp.float32),
    )(num, den.reshape(NP, 1), b.reshape(1, H))


def _tc_e_body(es_ref, ed_ref, at_ref, w_ref, wc_ref, b1_ref, w2_ref, b2_ref,
               out_ref):
    feats = jnp.concatenate([es_ref[...], ed_ref[...]], axis=1)
    hid = jnp.dot(feats.astype(jnp.bfloat16), w_ref[...].astype(jnp.bfloat16),
                  preferred_element_type=jnp.float32)
    at = at_ref[...]
    hid = hid + at[:, 0:1] * wc_ref[0:1, :] + at[:, 1:2] * wc_ref[1:2, :]
    hid = hid + b1_ref[...]
    hid = jnp.where(hid > 0.0, hid, jnp.exp(hid) - 1.0)
    val = jnp.sum(hid * w2_ref[...], axis=1, keepdims=True)
    out_ref[...] = val + b2_ref[...]


def _tc_e(es, ed, attr, Wm1, bm1, Wm2, bm2):
    BE = 2048
    nb = EPS // BE
    return pl.pallas_call(
        _tc_e_body,
        grid=(nb,),
        in_specs=[
            pl.BlockSpec((BE, H), lambda i: (i, 0)),
            pl.BlockSpec((BE, H), lambda i: (i, 0)),
            pl.BlockSpec((BE, 2), lambda i: (i, 0)),
            pl.BlockSpec((2 * H, EDGE_HID), lambda i: (0, 0)),
            pl.BlockSpec((2, EDGE_HID), lambda i: (0, 0)),
            pl.BlockSpec((1, EDGE_HID), lambda i: (0, 0)),
            pl.BlockSpec((1, EDGE_HID), lambda i: (0, 0)),
            pl.BlockSpec((1, 1), lambda i: (0, 0)),
        ],
        out_specs=pl.BlockSpec((BE, 1), lambda i: (i, 0)),
        out_shape=jax.ShapeDtypeStruct((EPS, 1), jnp.float32),
    )(es, ed, attr, Wm1[:2 * H], Wm1[2 * H:],
      bm1.reshape(1, EDGE_HID), Wm2.reshape(1, EDGE_HID),
      bm2.reshape(1, 1))


def kernel(x, edge_index, edge_attr, global_edge_index,
           W1, a_src1, a_dst1, b1, W2, a_src2, a_dst2, b2,
           Wm1, bm1, Wm2, bm2):
    x_pad = jnp.zeros((NP, DIN), jnp.float32).at[:N].set(x)
    padg = jnp.full((EGP - EG,), DUMMY, jnp.int32)
    srcg = jnp.concatenate([global_edge_index[0], padg])
    dstg = jnp.concatenate([global_edge_index[1], padg])
    pade = jnp.zeros((EP - E,), jnp.int32)
    ei0 = jnp.concatenate([edge_index[0], pade])
    ei1 = jnp.concatenate([edge_index[1], pade])
    attr_pad = jnp.zeros((EP, 2), jnp.float32).at[:E].set(edge_attr)

    ht1, ad1 = _tc_a(x_pad, W1, a_src1, a_dst1)
    num1, den1 = _gat_edge(ht1, srcg, dstg, ad1.reshape(NP))
    ht2, ad2 = _tc_c1(num1, den1, b1, W2, a_src2, a_dst2)
    num2, den2 = _gat_edge(ht2, srcg, dstg, ad2.reshape(NP))
    h2t = _tc_c2(num2, den2, b2)

    outs = []
    for i in range(NSLAB):
        lo = i * EPS
        es, ed = _edge_gather(h2t, lax.slice(ei0, (lo,), (lo + EPS,)),
                              lax.slice(ei1, (lo,), (lo + EPS,)))
        outs.append(_tc_e(es, ed,
                          lax.slice(attr_pad, (lo, 0), (lo + EPS, 2)),
                          Wm1, bm1, Wm2, bm2))
    return jnp.concatenate(outs, axis=0)[:E].reshape(E)


# speedup vs baseline: 1.3660x; 1.3660x over previous
import jax
import jax.numpy as jnp
from jax import lax
from jax.experimental import pallas as pl
from jax.experimental.pallas import tpu as pltpu
from jax.experimental.pallas import tpu_sc as plsc

N = 50000
E = 800000
EG = 300000
DIN = 3
H = 64
HH = 32
LW = 128
NC, NS = 2, 16
NW = NC * NS
EDGE_HID = 4 * (2 * H + 2)

NP = 51200
ROWS_PER_TILE = NP // NS
DUMMY = N

CH = 128
CG = 64
EG_CHUNKS = 294
EGP = NS * CG * EG_CHUNKS
EGT = CG * EG_CHUNKS

E_CHUNKS = 196
EP = NW * CH * E_CHUNKS
EPW = CH * E_CHUNKS

_MESH = plsc.VectorSubcoreMesh(core_axis_name="c", subcore_axis_name="s")
_SC_PARAMS = pltpu.CompilerParams(needs_layout_passes=False,
                                  use_tc_tiling_on_sc=False)


def _make_gat_edge_kernel():
    def body(htab, srcg, dstg, ad, num_out, den_out,
             src0, dst0, src1, dst1, ex_b, ad0, ad1, rows0, rows1, rows32_b,
             acc_sh, den_sh, ad_sh, semr0, semr1, sema0, sema1):
        c = lax.axis_index("c")
        s = lax.axis_index("s")
        r0 = s * ROWS_PER_TILE

        pltpu.sync_copy(ad.at[pl.ds(r0, ROWS_PER_TILE)],
                        ad_sh.at[pl.ds(r0, ROWS_PER_TILE)])

        zeros16 = jnp.zeros((16,), jnp.float32)
        for j in range(CG):
            rows32_b[j, pl.ds(0, 16)] = zeros16
            rows32_b[j, pl.ds(16, 16)] = zeros16
        for i in range(CG // 16):
            ex_b[pl.ds(i * 16, 16)] = zeros16
        nfull = ROWS_PER_TILE // CG
        for k in range(nfull):
            pltpu.sync_copy(rows32_b, acc_sh.at[pl.ds(r0 + k * CG, CG)])

        @pl.when(c == 0)
        def _zero_den():
            for k in range(nfull):
                pltpu.sync_copy(ex_b, den_sh.at[pl.ds(r0 + k * CG, CG)])

        plsc.subcore_barrier()

        ebase = s * EGT
        iota16 = lax.iota(jnp.int32, 16)
        col64 = jnp.full((16,), 2 * HH, jnp.int32)

        def fire(g, sb, db, rb, ab, semr, sema):
            e0 = ebase + g * CG
            pltpu.sync_copy(srcg.at[pl.ds(e0, CG)], sb)
            pltpu.sync_copy(dstg.at[pl.ds(e0, CG)], db)
            pltpu.async_copy(htab.at[sb], rb, semr)
            pltpu.async_copy(ad_sh.at[db], ab, sema)

        def process(sb, db, rb, ab, semr, sema):
            pltpu.make_async_copy(htab.at[sb], rb, semr).wait()
            pltpu.make_async_copy(ad_sh.at[db], ab, sema).wait()
            for i in range(CG // 16):
                av = plsc.load_gather(rb, [iota16 + (i * 16), col64])
                bv = ab[pl.ds(i * 16, 16)]
                e = av + bv
                e = jnp.where(e >= 0.0, e, 0.2 * e)
                ex_b[pl.ds(i * 16, 16)] = jnp.exp(e)

            def scale(off):
                for i in range(CG // 16):
                    ev = ex_b[pl.ds(i * 16, 16)]
                    for k in range(16):
                        j = i * 16 + k
                        m = ev[k]
                        rows32_b[j, pl.ds(0, 16)] = rb[j, pl.ds(off, 16)] * m
                        rows32_b[j, pl.ds(16, 16)] = \
                            rb[j, pl.ds(off + 16, 16)] * m

            @pl.when(c == 0)
            def _scale0():
                scale(0)

            @pl.when(c == 1)
            def _scale1():
                scale(HH)

            pltpu.sync_copy(rows32_b, acc_sh.at[db], add=True)

            @pl.when(c == 0)
            def _add_den():
                pltpu.sync_copy(ex_b, den_sh.at[db], add=True)

        fire(0, src0, dst0, rows0, ad0, semr0, sema0)

        def pair(t, carry):
            g0 = 2 * t
            fire(g0 + 1, src1, dst1, rows1, ad1, semr1, sema1)
            process(src0, dst0, rows0, ad0, semr0, sema0)
            g2 = jnp.where(g0 + 2 < EG_CHUNKS, g0 + 2, 0)
            fire(g2, src0, dst0, rows0, ad0, semr0, sema0)
            process(src1, dst1, rows1, ad1, semr1, sema1)
            return carry

        lax.fori_loop(0, EG_CHUNKS // 2, pair, 0)
        pltpu.make_async_copy(htab.at[src0], rows0, semr0).wait()
        pltpu.make_async_copy(ad_sh.at[dst0], ad0, sema0).wait()
        plsc.subcore_barrier()

        pltpu.sync_copy(acc_sh.at[pl.ds(r0, ROWS_PER_TILE)],
                        num_out.at[c, pl.ds(r0, ROWS_PER_TILE)])

        @pl.when(c == 0)
        def _write_den():
            pltpu.sync_copy(den_sh.at[pl.ds(r0, ROWS_PER_TILE)],
                            den_out.at[pl.ds(r0, ROWS_PER_TILE)])

    return pl.kernel(
        body,
        out_type=[
            jax.ShapeDtypeStruct((NC, NP, HH), jnp.float32),
            jax.ShapeDtypeStruct((NP,), jnp.float32),
        ],
        mesh=_MESH,
        scratch_types=[
            pltpu.VMEM((CG,), jnp.int32),
            pltpu.VMEM((CG,), jnp.int32),
            pltpu.VMEM((CG,), jnp.int32),
            pltpu.VMEM((CG,), jnp.int32),
            pltpu.VMEM((CG,), jnp.float32),
            pltpu.VMEM((CG,), jnp.float32),
            pltpu.VMEM((CG,), jnp.float32),
            pltpu.VMEM((CG, LW), jnp.float32),
            pltpu.VMEM((CG, LW), jnp.float32),
            pltpu.VMEM((CG, HH), jnp.float32),
            pltpu.VMEM_SHARED((NP, HH), jnp.float32),
            pltpu.VMEM_SHARED((NP,), jnp.float32),
            pltpu.VMEM_SHARED((NP,), jnp.float32),
            pltpu.SemaphoreType.DMA,
            pltpu.SemaphoreType.DMA,
            pltpu.SemaphoreType.DMA,
            pltpu.SemaphoreType.DMA,
        ],
        compiler_params=_SC_PARAMS,
    )


_gat_edge = _make_gat_edge_kernel()


def _make_edge_gather_kernel():
    def body(h2tab, ei0, ei1, es_out, ed_out,
             is0, id0, is1, id1, rs0, rd0, rs1, rd1,
             ss0, sd0, ss1, sd1):
        c = lax.axis_index("c")
        s = lax.axis_index("s")
        wid = s * NC + c
        base = wid * EPW

        def fire(g, ib, jb, rb, qb, sems, semd):
            e0 = base + g * CH
            pltpu.sync_copy(ei0.at[pl.ds(e0, CH)], ib)
            pltpu.sync_copy(ei1.at[pl.ds(e0, CH)], jb)
            pltpu.async_copy(h2tab.at[ib], rb, sems)
            pltpu.async_copy(h2tab.at[jb], qb, semd)

        def process(g, ib, jb, rb, qb, sems, semd):
            e0 = base + g * CH
            pltpu.make_async_copy(h2tab.at[ib], rb, sems).wait()
            pltpu.sync_copy(rb, es_out.at[pl.ds(e0, CH)])
            pltpu.make_async_copy(h2tab.at[jb], qb, semd).wait()
            pltpu.sync_copy(qb, ed_out.at[pl.ds(e0, CH)])

        fire(0, is0, id0, rs0, rd0, ss0, sd0)

        def pair(t, carry):
            g0 = 2 * t
            fire(g0 + 1, is1, id1, rs1, rd1, ss1, sd1)
            process(g0, is0, id0, rs0, rd0, ss0, sd0)
            g2 = jnp.where(g0 + 2 < E_CHUNKS, g0 + 2, 0)
            fire(g2, is0, id0, rs0, rd0, ss0, sd0)
            process(g0 + 1, is1, id1, rs1, rd1, ss1, sd1)
            return carry

        lax.fori_loop(0, E_CHUNKS // 2, pair, 0)
        pltpu.make_async_copy(h2tab.at[is0], rs0, ss0).wait()
        pltpu.make_async_copy(h2tab.at[id0], rd0, sd0).wait()

    return pl.kernel(
        body,
        out_type=[
            jax.ShapeDtypeStruct((EP, LW), jnp.float32),
            jax.ShapeDtypeStruct((EP, LW), jnp.float32),
        ],
        mesh=_MESH,
        scratch_types=[
            pltpu.VMEM((CH,), jnp.int32),
            pltpu.VMEM((CH,), jnp.int32),
            pltpu.VMEM((CH,), jnp.int32),
            pltpu.VMEM((CH,), jnp.int32),
            pltpu.VMEM((CH, LW), jnp.float32),
            pltpu.VMEM((CH, LW), jnp.float32),
            pltpu.VMEM((CH, LW), jnp.float32),
            pltpu.VMEM((CH, LW), jnp.float32),
            pltpu.SemaphoreType.DMA,
            pltpu.SemaphoreType.DMA,
            pltpu.SemaphoreType.DMA,
            pltpu.SemaphoreType.DMA,
        ],
        compiler_params=_SC_PARAMS,
    )


_edge_gather = _make_edge_gather_kernel()


def _tc_a_body(x_ref, w_ref, asr_ref, adr_ref, ht_ref, ad_ref):
    h = jnp.dot(x_ref[...], w_ref[...], preferred_element_type=jnp.float32)
    a_s = jnp.dot(h, asr_ref[...], preferred_element_type=jnp.float32)
    pad = jnp.zeros((h.shape[0], LW - H - 1), jnp.float32)
    ht_ref[...] = jnp.concatenate([h, a_s, pad], axis=1)
    ad_ref[...] = jnp.dot(h, adr_ref[...], preferred_element_type=jnp.float32)


def _tc_a(x_pad, W1, a_src, a_dst):
    BN = 6400
    nb = NP // BN
    return pl.pallas_call(
        _tc_a_body,
        grid=(nb,),
        in_specs=[
            pl.BlockSpec((BN, DIN), lambda i: (i, 0)),
            pl.BlockSpec((DIN, H), lambda i: (0, 0)),
            pl.BlockSpec((H, 1), lambda i: (0, 0)),
            pl.BlockSpec((H, 1), lambda i: (0, 0)),
        ],
        out_specs=[
            pl.BlockSpec((BN, LW), lambda i: (i, 0)),
            pl.BlockSpec((BN, 1), lambda i: (i, 0)),
        ],
        out_shape=[
            jax.ShapeDtypeStruct((NP, LW), jnp.float32),
            jax.ShapeDtypeStruct((NP, 1), jnp.float32),
        ],
    )(x_pad, W1, a_src.reshape(H, 1), a_dst.reshape(H, 1))


def _tc_c1_body(num_ref, den_ref, b_ref, w_ref, asr_ref, adr_ref,
                ht_ref, ad_ref):
    acc = jnp.concatenate([num_ref[0], num_ref[1]], axis=1)
    h = acc / (den_ref[...] + 1e-16) + b_ref[...]
    h = jnp.where(h > 0.0, h, jnp.exp(h) - 1.0)
    hlin = jnp.dot(h, w_ref[...], preferred_element_type=jnp.float32)
    a_s = jnp.dot(hlin, asr_ref[...], preferred_element_type=jnp.float32)
    pad = jnp.zeros((hlin.shape[0], LW - H - 1), jnp.float32)
    ht_ref[...] = jnp.concatenate([hlin, a_s, pad], axis=1)
    ad_ref[...] = jnp.dot(hlin, adr_ref[...], preferred_element_type=jnp.float32)


def _tc_c1(num, den, b, W2, a_src, a_dst):
    BN = 6400
    nb = NP // BN
    return pl.pallas_call(
        _tc_c1_body,
        grid=(nb,),
        in_specs=[
            pl.BlockSpec((2, BN, HH), lambda i: (0, i, 0)),
            pl.BlockSpec((BN, 1), lambda i: (i, 0)),
            pl.BlockSpec((1, H), lambda i: (0, 0)),
            pl.BlockSpec((H, H), lambda i: (0, 0)),
            pl.BlockSpec((H, 1), lambda i: (0, 0)),
            pl.BlockSpec((H, 1), lambda i: (0, 0)),
        ],
        out_specs=[
            pl.BlockSpec((BN, LW), lambda i: (i, 0)),
            pl.BlockSpec((BN, 1), lambda i: (i, 0)),
        ],
        out_shape=[
            jax.ShapeDtypeStruct((NP, LW), jnp.float32),
            jax.ShapeDtypeStruct((NP, 1), jnp.float32),
        ],
    )(num, den.reshape(NP, 1), b.reshape(1, H), W2,
      a_src.reshape(H, 1), a_dst.reshape(H, 1))


def _tc_c2_body(num_ref, den_ref, b_ref, ht_ref):
    acc = jnp.concatenate([num_ref[0], num_ref[1]], axis=1)
    h = acc / (den_ref[...] + 1e-16) + b_ref[...]
    h = jnp.where(h > 0.0, h, jnp.exp(h) - 1.0)
    pad = jnp.zeros((h.shape[0], LW - H), jnp.float32)
    ht_ref[...] = jnp.concatenate([h, pad], axis=1)


def _tc_c2(num, den, b):
    BN = 6400
    nb = NP // BN
    return pl.pallas_call(
        _tc_c2_body,
        grid=(nb,),
        in_specs=[
            pl.BlockSpec((2, BN, HH), lambda i: (0, i, 0)),
            pl.BlockSpec((BN, 1), lambda i: (i, 0)),
            pl.BlockSpec((1, H), lambda i: (0, 0)),
        ],
        out_specs=pl.BlockSpec((BN, LW), lambda i: (i, 0)),
        out_shape=jax.ShapeDtypeStruct((NP, LW), jnp.float32),
    )(num, den.reshape(NP, 1), b.reshape(1, H))


def _tc_e_body(es_ref, ed_ref, at_ref, w_ref, wc_ref, b1_ref, w2_ref, b2_ref,
               out_ref):
    feats = jnp.concatenate([es_ref[..., :H], ed_ref[..., :H]], axis=1)
    hid = jnp.dot(feats.astype(jnp.bfloat16), w_ref[...].astype(jnp.bfloat16),
                  preferred_element_type=jnp.float32)
    at = at_ref[...]
    hid = hid + at[:, 0:1] * wc_ref[0:1, :] + at[:, 1:2] * wc_ref[1:2, :]
    hid = hid + b1_ref[...]
    hid = jnp.where(hid > 0.0, hid, jnp.exp(hid) - 1.0)
    val = jnp.sum(hid * w2_ref[...], axis=1, keepdims=True)
    out_ref[...] = val + b2_ref[...]


def _tc_e(es, ed, attr, Wm1, bm1, Wm2, bm2):
    BE = 2048
    nb = pl.cdiv(E, BE)
    return pl.pallas_call(
        _tc_e_body,
        grid=(nb,),
        in_specs=[
            pl.BlockSpec((BE, LW), lambda i: (i, 0)),
            pl.BlockSpec((BE, LW), lambda i: (i, 0)),
            pl.BlockSpec((BE, 2), lambda i: (i, 0)),
            pl.BlockSpec((2 * H, EDGE_HID), lambda i: (0, 0)),
            pl.BlockSpec((2, EDGE_HID), lambda i: (0, 0)),
            pl.BlockSpec((1, EDGE_HID), lambda i: (0, 0)),
            pl.BlockSpec((1, EDGE_HID), lambda i: (0, 0)),
            pl.BlockSpec((1, 1), lambda i: (0, 0)),
        ],
        out_specs=pl.BlockSpec((BE, 1), lambda i: (i, 0)),
        out_shape=jax.ShapeDtypeStruct((E, 1), jnp.float32),
    )(es, ed, attr, Wm1[:2 * H], Wm1[2 * H:],
      bm1.reshape(1, EDGE_HID), Wm2.reshape(1, EDGE_HID),
      bm2.reshape(1, 1))


def kernel(x, edge_index, edge_attr, global_edge_index,
           W1, a_src1, a_dst1, b1, W2, a_src2, a_dst2, b2,
           Wm1, bm1, Wm2, bm2):
    x_pad = jnp.zeros((NP, DIN), jnp.float32).at[:N].set(x)
    padg = jnp.full((EGP - EG,), DUMMY, jnp.int32)
    srcg = jnp.concatenate([global_edge_index[0], padg])
    dstg = jnp.concatenate([global_edge_index[1], padg])
    pade = jnp.zeros((EP - E,), jnp.int32)
    ei0 = jnp.concatenate([edge_index[0], pade])
    ei1 = jnp.concatenate([edge_index[1], pade])

    ht1, ad1 = _tc_a(x_pad, W1, a_src1, a_dst1)
    num1, den1 = _gat_edge(ht1, srcg, dstg, ad1.reshape(NP))
    ht2, ad2 = _tc_c1(num1, den1, b1, W2, a_src2, a_dst2)
    num2, den2 = _gat_edge(ht2, srcg, dstg, ad2.reshape(NP))
    h2t = _tc_c2(num2, den2, b2)

    es, ed = _edge_gather(h2t, ei0, ei1)
    out = _tc_e(es, ed, edge_attr, Wm1, bm1, Wm2, bm2)
    return out.reshape(E)
